# baseline (speedup 1.0000x reference)
import functools

import jax
import jax.numpy as jnp
from jax.experimental import pallas as pl
from jax.experimental.pallas import tpu as pltpu

_CL = 8
_V = 1024
_DZ = 256
_B = 4096
_R = _B * _CL
_BB = 128
_GRID = _B // _BB
_EPS = 1e-06
_INV_LN2 = 1.4426950408889634


def _vq_block(q_ref, cb_ref, lat_ref, emb_ref, msg_ref, rep_ref, part_ref,
              cb2_ref):
    i = pl.program_id(0)
    cb = cb_ref[...]

    @pl.when(i == 0)
    def _prep():
        cb2_ref[0, :] = jnp.sum(cb * cb, axis=1)

    cb2 = cb2_ref[0, :]

    pyp = jnp.zeros((1, _V), jnp.float32)
    hxp = jnp.float32(0.0)
    y_hards = []
    quants = []
    codes = []
    for l in range(_CL):
        x = q_ref[:, l * _DZ:(l + 1) * _DZ]
        lat_ref[:, l, :] = x
        lat2 = jnp.sum(x * x, axis=1, keepdims=True)
        cross = jax.lax.dot_general(
            x, cb, (((1,), (1,)), ((), ())),
            preferred_element_type=jnp.float32)
        scores = -0.5 * (lat2 - 2.0 * cross + cb2[None, :])
        m = jnp.max(scores, axis=1, keepdims=True)
        d = scores - m
        e = jnp.exp(d)
        s = jnp.sum(e, axis=1, keepdims=True)
        w = 1.0 / s

        col = jax.lax.broadcasted_iota(jnp.int32, (_BB, _V), 1)
        cand = jnp.where(scores == m, col, _V)
        cmin = jnp.min(cand, axis=1, keepdims=True)
        codes.append(jnp.min(cand, axis=1).reshape(_BB, 1))
        y_hard = (cand == cmin).astype(jnp.float32)
        y_hards.append(y_hard)
        quants.append(jax.lax.dot_general(
            y_hard, cb, (((1,), (0,)), ((), ())),
            preferred_element_type=jnp.float32))

        pyp = pyp + jax.lax.dot_general(
            w.reshape(1, _BB), e, (((1,), (0,)), ((), ())),
            preferred_element_type=jnp.float32)
        t = jnp.sum(e * d, axis=1, keepdims=True)
        hrow = (w[:, 0] * t[:, 0] - jnp.log(s[:, 0])) * _INV_LN2
        hxp = hxp + jnp.sum(hrow)

    msg_ref[...] = jnp.concatenate(codes, axis=1)
    rep_ref[...] = jnp.concatenate(y_hards, axis=1)
    emb_ref[...] = jnp.concatenate(quants, axis=1)
    part_ref[...] = jnp.concatenate(
        [pyp, jnp.full((1, _V), hxp, jnp.float32)], axis=1
    ).reshape(1, 1, 2 * _V)


def _stats_block(part_ref, stats_ref):
    p = part_ref[...].reshape(_GRID, 2 * _V)
    py = jnp.sum(p[:, :_V], axis=0) * (1.0 / _R)
    hy = -jnp.sum(py * jnp.log2(py + _EPS))
    hyx = -jnp.sum(p[:, _V:_V + 128], axis=0)[0] * (1.0 / _R)
    lane = jax.lax.broadcasted_iota(jnp.int32, (8, 128), 1)
    stats_ref[...] = jnp.where(lane == 0, hy, hyx)


@jax.jit
def _vq_call(query, codebook):
    latent, emb, msg, rep, part = pl.pallas_call(
        _vq_block,
        grid=(_GRID,),
        in_specs=[
            pl.BlockSpec((_BB, _CL * _DZ), lambda i: (i, 0)),
            pl.BlockSpec((_V, _DZ), lambda i: (0, 0)),
        ],
        out_specs=[
            pl.BlockSpec((_BB, _CL, _DZ), lambda i: (i, 0, 0)),
            pl.BlockSpec((_BB, _CL * _DZ), lambda i: (i, 0)),
            pl.BlockSpec((_BB, _CL), lambda i: (i, 0)),
            pl.BlockSpec((_BB, _CL * _V), lambda i: (i, 0)),
            pl.BlockSpec((1, 1, 2 * _V), lambda i: (i, 0, 0)),
        ],
        out_shape=[
            jax.ShapeDtypeStruct((_B, _CL, _DZ), jnp.float32),
            jax.ShapeDtypeStruct((_B, _CL * _DZ), jnp.float32),
            jax.ShapeDtypeStruct((_B, _CL), jnp.int32),
            jax.ShapeDtypeStruct((_B, _CL * _V), jnp.float32),
            jax.ShapeDtypeStruct((_GRID, 1, 2 * _V), jnp.float32),
        ],
        scratch_shapes=[pltpu.VMEM((1, _V), jnp.float32)],
        compiler_params=pltpu.CompilerParams(
            dimension_semantics=("arbitrary",)),
    )(query, codebook)
    stats = pl.pallas_call(
        _stats_block,
        out_shape=jax.ShapeDtypeStruct((8, 128), jnp.float32),
    )(part)
    return latent, emb, msg, rep, stats


def kernel(query, codebook):
    latent, emb, msg, rep, stats = _vq_call(query, codebook)
    hy = stats[0, 0]
    hyx = stats[0, 1]
    loss = jnp.float32(0.0)
    return (latent, emb, msg, rep, hy, hyx, loss)


# speedup vs baseline: 1.1929x; 1.1929x over previous
import functools

import jax
import jax.numpy as jnp
from jax.experimental import pallas as pl
from jax.experimental.pallas import tpu as pltpu

_CL = 8
_V = 1024
_DZ = 256
_B = 4096
_R = _B * _CL
_BB = 128
_RB = _BB * _CL
_GRID = _B // _BB
_EPS = 1e-06
_INV_LN2 = 1.4426950408889634


def _vq_block(q_ref, cb_ref, lat_ref, emb_ref, msg_ref, rep_ref, part_ref,
              cb2_ref):
    i = pl.program_id(0)
    cb = cb_ref[...]

    @pl.when(i == 0)
    def _prep():
        cb2_ref[0, :] = jnp.sum(cb * cb, axis=1)

    cb2 = cb2_ref[0, :]
    x = q_ref[...].reshape(_RB, _DZ)
    lat_ref[...] = x.reshape(_BB, _CL, _DZ)
    lat2 = jnp.sum(x * x, axis=1, keepdims=True)
    cross = jax.lax.dot_general(
        x, cb, (((1,), (1,)), ((), ())),
        preferred_element_type=jnp.float32)
    scores = -0.5 * (lat2 - 2.0 * cross + cb2[None, :])
    m = jnp.max(scores, axis=1, keepdims=True)
    d = scores - m
    e = jnp.exp(d)
    s = jnp.sum(e, axis=1, keepdims=True)
    w = 1.0 / s

    col = jax.lax.broadcasted_iota(jnp.int32, (_RB, _V), 1)
    cand = jnp.where(scores == m, col, _V)
    cmin = jnp.min(cand, axis=1, keepdims=True)
    code = jnp.min(cand, axis=1)
    msg_ref[...] = code.reshape(_BB, _CL)

    y_hard = (cand == cmin).astype(jnp.float32)
    rep_ref[...] = y_hard.reshape(_BB, _CL * _V)
    quant = jax.lax.dot_general(
        y_hard, cb, (((1,), (0,)), ((), ())),
        preferred_element_type=jnp.float32)
    emb_ref[...] = quant.reshape(_BB, _CL * _DZ)

    pyp = jax.lax.dot_general(
        w.reshape(1, _RB), e, (((1,), (0,)), ((), ())),
        preferred_element_type=jnp.float32)

    t = jnp.sum(e * d, axis=1, keepdims=True)
    hrow = (w[:, 0] * t[:, 0] - jnp.log(s[:, 0])) * _INV_LN2
    hxp = jnp.sum(hrow)
    part_ref[...] = jnp.concatenate(
        [pyp, jnp.full((1, _V), hxp, jnp.float32)], axis=1
    ).reshape(1, 1, 2 * _V)


def _stats_block(part_ref, stats_ref):
    p = part_ref[...].reshape(_GRID, 2 * _V)
    py = jnp.sum(p[:, :_V], axis=0) * (1.0 / _R)
    hy = -jnp.sum(py * jnp.log2(py + _EPS))
    hyx = -jnp.sum(p[:, _V:_V + 128], axis=0)[0] * (1.0 / _R)
    lane = jax.lax.broadcasted_iota(jnp.int32, (8, 128), 1)
    stats_ref[...] = jnp.where(lane == 0, hy, hyx)


@jax.jit
def _vq_call(query, codebook):
    latent, emb, msg, rep, part = pl.pallas_call(
        _vq_block,
        grid=(_GRID,),
        in_specs=[
            pl.BlockSpec((_BB, _CL * _DZ), lambda i: (i, 0)),
            pl.BlockSpec((_V, _DZ), lambda i: (0, 0)),
        ],
        out_specs=[
            pl.BlockSpec((_BB, _CL, _DZ), lambda i: (i, 0, 0)),
            pl.BlockSpec((_BB, _CL * _DZ), lambda i: (i, 0)),
            pl.BlockSpec((_BB, _CL), lambda i: (i, 0)),
            pl.BlockSpec((_BB, _CL * _V), lambda i: (i, 0)),
            pl.BlockSpec((1, 1, 2 * _V), lambda i: (i, 0, 0)),
        ],
        out_shape=[
            jax.ShapeDtypeStruct((_B, _CL, _DZ), jnp.float32),
            jax.ShapeDtypeStruct((_B, _CL * _DZ), jnp.float32),
            jax.ShapeDtypeStruct((_B, _CL), jnp.int32),
            jax.ShapeDtypeStruct((_B, _CL * _V), jnp.float32),
            jax.ShapeDtypeStruct((_GRID, 1, 2 * _V), jnp.float32),
        ],
        scratch_shapes=[pltpu.VMEM((1, _V), jnp.float32)],
        compiler_params=pltpu.CompilerParams(
            dimension_semantics=("arbitrary",)),
    )(query, codebook)
    stats = pl.pallas_call(
        _stats_block,
        out_shape=jax.ShapeDtypeStruct((8, 128), jnp.float32),
    )(part)
    return latent, emb, msg, rep, stats


def kernel(query, codebook):
    latent, emb, msg, rep, stats = _vq_call(query, codebook)
    hy = stats[0, 0]
    hyx = stats[0, 1]
    loss = jnp.float32(0.0)
    return (latent, emb, msg, rep, hy, hyx, loss)
